# baseline (speedup 1.0000x reference)
import functools

import jax
import jax.numpy as jnp
from jax import lax
from jax.experimental import pallas as pl
from jax.experimental.pallas import tpu as pltpu
from jax.experimental.pallas import tpu_sc as plsc

B = 16384
D = 32
V = 100001
VPAD = 100096
NUM_CORES = 2
NUM_SUBCORES = 16
NW = NUM_CORES * NUM_SUBCORES
N_TILES = VPAD // 128
SLAB_TILES = 25
SLAB_COLS = SLAB_TILES * 128
N_BANDS = D // 8
POS_SHIFT = 15
N_STRIPES = 8
BLOCK = 2048
N_BLOCKS = B // BLOCK
GROUPS_PER_CHAIN = BLOCK // (16 * N_STRIPES)
STRIPE_CAP = BLOCK // N_STRIPES + 16
HITS_CAP = N_STRIPES * STRIPE_CAP
CHUNK = 128
DUMP_ROW = B


def _make_kernel():
    mesh = plsc.VectorSubcoreMesh(core_axis_name="c", subcore_axis_name="s")

    @functools.partial(
        pl.kernel,
        mesh=mesh,
        out_type=jax.ShapeDtypeStruct((B + 1, 128), jnp.float32),
        scratch_types=[
            pltpu.VMEM((2 * BLOCK,), jnp.int32),
            pltpu.VMEM((N_BANDS, 8, SLAB_COLS), jnp.float32),
            pltpu.VMEM((HITS_CAP,), jnp.int32),
            pltpu.VMEM((CHUNK, 128), jnp.float32),
            pltpu.VMEM((CHUNK,), jnp.int32),
            pltpu.SemaphoreType.DMA,
            pltpu.SemaphoreType.DMA,
            pltpu.SemaphoreType.DMA,
        ],
        compiler_params=pltpu.CompilerParams(
            use_tc_tiling_on_sc=True, needs_layout_passes=False
        ),
    )
    def k(
        tbl_hbm, idx_hbm, out_hbm, idx_v, slab_v, hits_v, rows_v, pos_v,
        sem, osem, isem,
    ):
        wid = lax.axis_index("s") * NUM_CORES + lax.axis_index("c")
        small = jnp.int32(25 * 14)
        t0 = jnp.where(wid < 14, 25 * wid, small + 24 * (wid - 14))
        ntc = jnp.where(wid < 14, 25, 24)
        slab_t0 = jnp.minimum(t0, N_TILES - SLAB_TILES)
        slab_c0 = slab_t0 * 128
        sel_a = t0 * 128
        sel_b = (t0 + ntc) * 128

        slab_cps = []
        for band in range(N_BANDS):
            slab_cps.append(
                pltpu.async_copy(
                    tbl_hbm.at[pl.ds(band * 8, 8), pl.ds(slab_c0, SLAB_COLS)],
                    slab_v.at[band],
                    sem,
                )
            )

        lane = lax.iota(jnp.int32, 16)
        zeros = jnp.full((16,), 0, jnp.int32)
        sel_a_v = zeros + sel_a
        sel_b_v = zeros + sel_b
        c0_v = zeros + slab_c0
        pos_mask = zeros + ((1 << POS_SHIFT) - 1)
        dump_v = zeros + DUMP_ROW

        idx_cp = pltpu.async_copy(
            idx_hbm.at[pl.ds(0, BLOCK)], idx_v.at[pl.ds(0, BLOCK)], isem
        )
        for blk in range(N_BLOCKS):
            ibase = (blk % 2) * BLOCK
            idx_cp.wait()
            if blk + 1 < N_BLOCKS:
                idx_cp = pltpu.async_copy(
                    idx_hbm.at[pl.ds((blk + 1) * BLOCK, BLOCK)],
                    idx_v.at[pl.ds(((blk + 1) % 2) * BLOCK, BLOCK)],
                    isem,
                )

            def sel_body(j, offs, blk=blk, ibase=ibase):
                new_offs = []
                for s in range(N_STRIPES):
                    local0 = (j * N_STRIPES + s) * 16
                    vec = idx_v[pl.ds(ibase + local0, 16)]
                    m = jnp.logical_and(vec >= sel_a_v, vec < sel_b_v)
                    cnt = plsc.all_reduce_population_count(m)
                    packed = (
                        (lane + (blk * BLOCK + local0))
                        + lax.shift_left(vec - c0_v, POS_SHIFT)
                    )
                    plsc.store_compressed(
                        hits_v.at[pl.ds(s * STRIPE_CAP + offs[s], 16)],
                        packed,
                        mask=m,
                    )
                    new_offs.append(offs[s] + cnt[0])
                return tuple(new_offs)

            offs = lax.fori_loop(
                0, GROUPS_PER_CHAIN, sel_body, (jnp.int32(0),) * N_STRIPES
            )

            cum = offs[0]
            for s in range(1, N_STRIPES):
                cnt = offs[s]

                def merge_body(t, cum, s=s):
                    hits_v[pl.ds(cum + t * 16, 16)] = hits_v[
                        pl.ds(s * STRIPE_CAP + t * 16, 16)
                    ]
                    return cum

                lax.fori_loop(
                    0, lax.shift_right_logical(cnt + 15, 4), merge_body, cum
                )
                cum = cum + cnt

            for kk in range(CHUNK // 16):
                hits_v[pl.ds(cum + kk * 16, 16)] = dump_v

            if blk == 0:
                for c in slab_cps:
                    c.wait()

            n_chunks = lax.shift_right_logical(cum + CHUNK - 1, 7)

            def ext_body(ch, carry):
                def grp_body(g, carry, ch=ch):
                    pk = hits_v[pl.ds(ch * CHUNK + g * 16, 16)]
                    pos = pk & pos_mask
                    col = lax.shift_right_logical(pk, POS_SHIFT)
                    pos_v[pl.ds(g * 16, 16)] = pos
                    row_idx = lane + g * 16
                    for d in range(D):
                        vals = plsc.load_gather(
                            slab_v, [zeros + (d // 8), zeros + (d % 8), col]
                        )
                        plsc.store_scatter(rows_v, [row_idx, zeros + d], vals)
                    return carry

                lax.fori_loop(0, CHUNK // 16, grp_body, jnp.int32(0))
                pltpu.async_copy(rows_v, out_hbm.at[pos_v], osem).wait()
                return carry

            lax.fori_loop(0, n_chunks, ext_body, jnp.int32(0))

    return k


@jax.jit
def kernel(template_table, eventids):
    idx = eventids.astype(jnp.int32)
    tbl_t = template_table.T
    out128 = _make_kernel()(tbl_t, idx)
    return out128[:B, :D]


# speedup vs baseline: 10.4421x; 10.4421x over previous
import functools

import jax
import jax.numpy as jnp
from jax import lax
from jax.experimental import pallas as pl
from jax.experimental.pallas import tpu as pltpu
from jax.experimental.pallas import tpu_sc as plsc

B = 16384
D = 32
V = 100001
VPAD = 100096
NUM_CORES = 2
NUM_SUBCORES = 16
NW = NUM_CORES * NUM_SUBCORES
N_TILES = VPAD // 128
SLAB_TILES = 25
SLAB_COLS = SLAB_TILES * 128
N_BANDS = D // 8
IDX_BLK = 4096
POS_SHIFT = 14


def _make_kernel():
    mesh = plsc.VectorSubcoreMesh(core_axis_name="c", subcore_axis_name="s")

    @functools.partial(
        pl.kernel,
        mesh=mesh,
        out_type=jax.ShapeDtypeStruct((B * D,), jnp.float32),
        scratch_types=[
            pltpu.VMEM((IDX_BLK,), jnp.int32),
            pltpu.VMEM((N_BANDS, 8, SLAB_COLS), jnp.float32),
            pltpu.VMEM((B + 16,), jnp.int32),
            pltpu.VMEM((16, D), jnp.float32),
            pltpu.SemaphoreType.DMA,
            pltpu.SemaphoreType.DMA,
        ],
        compiler_params=pltpu.CompilerParams(
            use_tc_tiling_on_sc=True, needs_layout_passes=False
        ),
    )
    def k(tbl_hbm, idx_hbm, out_hbm, idx_v, slab_v, hits_v, ring_v, sem, osem):
        wid = lax.axis_index("s") * NUM_CORES + lax.axis_index("c")
        small = jnp.int32(25 * 14)
        t0 = jnp.where(wid < 14, 25 * wid, small + 24 * (wid - 14))
        ntc = jnp.where(wid < 14, 25, 24)
        slab_t0 = jnp.minimum(t0, N_TILES - SLAB_TILES)
        slab_c0 = slab_t0 * 128
        sel_a = t0 * 128
        sel_b = (t0 + ntc) * 128

        slab_cps = []
        for band in range(N_BANDS):
            slab_cps.append(
                pltpu.async_copy(
                    tbl_hbm.at[pl.ds(band * 8, 8), pl.ds(slab_c0, SLAB_COLS)],
                    slab_v.at[band],
                    sem,
                )
            )

        lane = lax.iota(jnp.int32, 16)
        zeros = jnp.full((16,), 0, jnp.int32)
        sel_a_v = zeros + sel_a
        sel_b_v = zeros + sel_b
        c0_v = zeros + slab_c0

        total = jnp.int32(0)
        for blk in range(B // IDX_BLK):
            pltpu.sync_copy(idx_hbm.at[pl.ds(blk * IDX_BLK, IDX_BLK)], idx_v)

            def sel_body(g, off, blk=blk):
                vec = idx_v[pl.ds(g * 16, 16)]
                m = jnp.logical_and(vec >= sel_a_v, vec < sel_b_v)
                cnt = plsc.all_reduce_population_count(m)
                pos_v = lane + (blk * IDX_BLK + g * 16)
                packed = pos_v + lax.shift_left(vec - c0_v, POS_SHIFT)
                plsc.store_compressed(hits_v.at[pl.ds(off, 16)], packed, mask=m)
                return off + cnt[0]

            total = lax.fori_loop(0, IDX_BLK // 16, sel_body, total)

        first_vec = hits_v[pl.ds(0, 16)]
        first = zeros + first_vec[0]

        @pl.when(total > 0)
        def _():
            hits_v[pl.ds(total, 16)] = first

        for c in slab_cps:
            c.wait()

        band_idx, sub_idx = [], []
        for h in range(2):
            d = lane + h * 16
            band_idx.append(lax.shift_right_logical(d, 3))
            sub_idx.append(d & 7)
        pos_mask = zeros + ((1 << POS_SHIFT) - 1)
        n_grp = lax.shift_right_logical(total + 15, 4)

        def ext_body(eg, carry):
            pk = hits_v[pl.ds(eg * 16, 16)]
            pos_v = pk & pos_mask
            col_v = lax.shift_right_logical(pk, POS_SHIFT)
            cps = []
            for e in range(16):
                col = zeros + col_v[e]
                for h in range(2):
                    ring_v[e, pl.ds(h * 16, 16)] = plsc.load_gather(
                        slab_v, [band_idx[h], sub_idx[h], col]
                    )
                cps.append(
                    pltpu.async_copy(
                        ring_v.at[e],
                        out_hbm.at[pl.ds(pos_v[e] * D, D)],
                        osem,
                    )
                )
            for c in cps:
                c.wait()
            return carry

        lax.fori_loop(0, n_grp, ext_body, jnp.int32(0))

    return k


@jax.jit
def kernel(template_table, eventids):
    idx = eventids.astype(jnp.int32)
    tbl_t = template_table.T
    out1d = _make_kernel()(tbl_t, idx)
    return out1d.reshape(B, D)


# speedup vs baseline: 10.7696x; 1.0314x over previous
import functools

import jax
import jax.numpy as jnp
from jax import lax
from jax.experimental import pallas as pl
from jax.experimental.pallas import tpu as pltpu
from jax.experimental.pallas import tpu_sc as plsc

B = 16384
D = 32
V = 100001
VPAD = 100096
NUM_CORES = 2
NUM_SUBCORES = 16
NW = NUM_CORES * NUM_SUBCORES
N_TILES = VPAD // 128
SLAB_TILES = 25
SLAB_COLS = SLAB_TILES * 128
N_BANDS = D // 8
IDX_BLK = 4096
POS_SHIFT = 14


def _make_kernel():
    mesh = plsc.VectorSubcoreMesh(core_axis_name="c", subcore_axis_name="s")

    @functools.partial(
        pl.kernel,
        mesh=mesh,
        out_type=jax.ShapeDtypeStruct((B * D,), jnp.float32),
        scratch_types=[
            pltpu.VMEM((2 * IDX_BLK,), jnp.int32),
            pltpu.VMEM((N_BANDS, 8, SLAB_COLS), jnp.float32),
            pltpu.VMEM((B + 16,), jnp.int32),
            pltpu.VMEM((16, D), jnp.float32),
            pltpu.VMEM((16 * D,), jnp.float32),
            pltpu.SemaphoreType.DMA,
            pltpu.SemaphoreType.DMA,
            pltpu.SemaphoreType.DMA,
        ],
        compiler_params=pltpu.CompilerParams(
            use_tc_tiling_on_sc=True, needs_layout_passes=False
        ),
    )
    def k(
        tbl_hbm, idx_hbm, out_hbm, idx_v, slab_v, hits_v, ring_v, drain_v,
        sem, osem, isem,
    ):
        wid = lax.axis_index("s") * NUM_CORES + lax.axis_index("c")
        small = jnp.int32(25 * 14)
        t0 = jnp.where(wid < 14, 25 * wid, small + 24 * (wid - 14))
        ntc = jnp.where(wid < 14, 25, 24)
        slab_t0 = jnp.minimum(t0, N_TILES - SLAB_TILES)
        slab_c0 = slab_t0 * 128
        sel_a = t0 * 128
        sel_b = (t0 + ntc) * 128

        slab_cps = []
        for band in range(N_BANDS):
            slab_cps.append(
                pltpu.async_copy(
                    tbl_hbm.at[pl.ds(band * 8, 8), pl.ds(slab_c0, SLAB_COLS)],
                    slab_v.at[band],
                    sem,
                )
            )

        lane = lax.iota(jnp.int32, 16)
        zeros = jnp.full((16,), 0, jnp.int32)
        sel_a_v = zeros + sel_a
        sel_b_v = zeros + sel_b
        c0_v = zeros + slab_c0

        total = jnp.int32(0)
        idx_cp = pltpu.async_copy(
            idx_hbm.at[pl.ds(0, IDX_BLK)], idx_v.at[pl.ds(0, IDX_BLK)], isem
        )
        for blk in range(B // IDX_BLK):
            idx_cp.wait()
            if blk + 1 < B // IDX_BLK:
                idx_cp = pltpu.async_copy(
                    idx_hbm.at[pl.ds((blk + 1) * IDX_BLK, IDX_BLK)],
                    idx_v.at[pl.ds(((blk + 1) % 2) * IDX_BLK, IDX_BLK)],
                    isem,
                )
            ibase = (blk % 2) * IDX_BLK

            def sel_body(g, off, blk=blk, ibase=ibase):
                vec = idx_v[pl.ds(ibase + g * 16, 16)]
                m = jnp.logical_and(vec >= sel_a_v, vec < sel_b_v)
                cnt = plsc.all_reduce_population_count(m)
                pos_v = lane + (blk * IDX_BLK + g * 16)
                packed = pos_v + lax.shift_left(vec - c0_v, POS_SHIFT)
                plsc.store_compressed(hits_v.at[pl.ds(off, 16)], packed, mask=m)
                return off + cnt[0]

            total = lax.fori_loop(0, IDX_BLK // 16, sel_body, total)

        first_vec = hits_v[pl.ds(0, 16)]
        first = zeros + first_vec[0]

        @pl.when(total > 0)
        def _():
            hits_v[pl.ds(total, 16)] = first

        for c in slab_cps:
            c.wait()

        band_idx, sub_idx = [], []
        for h in range(2):
            d = lane + h * 16
            band_idx.append(lax.shift_right_logical(d, 3))
            sub_idx.append(d & 7)
        pos_mask = zeros + ((1 << POS_SHIFT) - 1)
        n_grp = lax.shift_right_logical(total + 15, 4)

        def ext_body(eg, carry):
            pk = hits_v[pl.ds(eg * 16, 16)]
            pos_v = pk & pos_mask
            col_v = lax.shift_right_logical(pk, POS_SHIFT)
            for e in range(16):
                col = zeros + col_v[e]
                for h in range(2):
                    ring_v[e, pl.ds(h * 16, 16)] = plsc.load_gather(
                        slab_v, [band_idx[h], sub_idx[h], col]
                    )
                pltpu.async_copy(
                    ring_v.at[e],
                    out_hbm.at[pl.ds(pos_v[e] * D, D)],
                    osem,
                )
            pltpu.make_async_copy(
                out_hbm.at[pl.ds(0, 16 * D)], drain_v, osem
            ).wait()
            return carry

        lax.fori_loop(0, n_grp, ext_body, jnp.int32(0))

    return k


@jax.jit
def kernel(template_table, eventids):
    idx = eventids.astype(jnp.int32)
    tbl_t = template_table.T
    out1d = _make_kernel()(tbl_t, idx)
    return out1d.reshape(B, D)
